# baseline (speedup 1.0000x reference)
import functools

import jax
import jax.numpy as jnp
from jax import lax
from jax.experimental import pallas as pl
from jax.experimental.pallas import tpu as pltpu
from jax.experimental.pallas import tpu_sc as plsc

D = 32
DPAD = 128
NI = 4096
NJ = 200
NC, NS = 2, 16
NW = NC * NS
SLAB = NI // NW
NBUF = 4
L = 16

_mesh = plsc.VectorSubcoreMesh(core_axis_name="c", subcore_axis_name="s")


@functools.partial(
    pl.kernel,
    out_type=jax.ShapeDtypeStruct((NJ, D, NI), jnp.float32),
    mesh=_mesh,
    scratch_types=(
        [pltpu.VMEM((NJ, SLAB), jnp.int32)]
        + [pltpu.VMEM((SLAB, DPAD), jnp.float32) for _ in range(NBUF)]
        + [pltpu.VMEM((D, SLAB), jnp.float32) for _ in range(NBUF)]
        + [pltpu.SemaphoreType.DMA for _ in range(2 * NBUF)]
    ),
    compiler_params=pltpu.CompilerParams(needs_layout_passes=False),
)
def _embed_gather(ids_hbm, table_hbm, out_hbm, ids_v, *rest):
    rows = rest[:NBUF]
    outs = rest[NBUF:2 * NBUF]
    gsems = rest[2 * NBUF:3 * NBUF]
    ssems = rest[3 * NBUF:]
    wid = lax.axis_index("s") * NC + lax.axis_index("c")
    base_i = wid * SLAB

    pltpu.sync_copy(ids_hbm.at[:, pl.ds(base_i, SLAB)], ids_v)

    def fire_gather(j, b):
        pltpu.async_copy(table_hbm.at[ids_v.at[j]], rows[b], gsems[b])

    def wait_gather(b):
        pltpu.make_async_copy(table_hbm.at[pl.ds(0, SLAB)], rows[b], gsems[b]).wait()

    def fire_store(j, b):
        pltpu.async_copy(outs[b], out_hbm.at[j, :, pl.ds(base_i, SLAB)], ssems[b])

    def wait_store(b):
        pltpu.make_async_copy(outs[b], out_hbm.at[0, :, pl.ds(0, SLAB)], ssems[b]).wait()

    ivecs = [lax.iota(jnp.int32, L) + m * L for m in range(SLAB // L)]

    def extract(b):
        @pl.loop(0, D)
        def _per_k(k):
            kvec = jnp.zeros((L,), jnp.int32) + k
            for m in range(SLAB // L):
                x = plsc.load_gather(rows[b], [ivecs[m], kvec])
                outs[b][k, pl.ds(m * L, L)] = x

    for b in range(NBUF):
        fire_gather(b, b)
    for b in range(NBUF):
        wait_gather(b)
        extract(b)
        fire_store(b, b)
        fire_gather(b + NBUF, b)

    @pl.loop(NBUF, NJ - NBUF, step=NBUF)
    def _pipeline(j0):
        for b in range(NBUF):
            j = j0 + b
            wait_gather(b)
            wait_store(b)
            extract(b)
            fire_store(j, b)
            fire_gather(j + NBUF, b)

    for b in range(NBUF):
        wait_gather(b)
        wait_store(b)
        extract(b)
        fire_store(NJ - NBUF + b, b)
    for b in range(NBUF):
        wait_store(b)


def kernel(input_ids, table):
    ids_t = input_ids.T
    tbl128 = jnp.pad(table, ((0, 0), (0, DPAD - D)))
    out_t = _embed_gather(ids_t, tbl128)
    return out_t.transpose(2, 0, 1)
